# baseline (speedup 1.0000x reference)
import functools

import jax
import jax.numpy as jnp
from jax import lax
from jax.experimental import pallas as pl
from jax.experimental.pallas import tpu as pltpu
from jax.experimental.pallas import tpu_sc as plsc

_N_EDGES = 65536
_D = 1024
_NC = 2
_NS = 16
_NW = _NC * _NS
_EPW = _N_EDGES // _NW
_C = 8
_NCHUNK = _EPW // _C
_NBUF = 4
_NSLICE = _D // 16


def _body(x_hbm, idx_hbm, rel_hbm, out_hbm,
          idx_v, rel_v, bufs, scores_v, sems):
    wid = lax.axis_index("s") * _NC + lax.axis_index("c")

    pltpu.sync_copy(idx_hbm.at[wid], idx_v)
    pltpu.sync_copy(rel_hbm, rel_v)

    def start(c, buf, sem):
        pltpu.make_async_copy(x_hbm.at[idx_v.at[c]], buf, sem).start()

    def drain(buf, sem):
        pltpu.make_async_copy(x_hbm.at[idx_v.at[0]], buf, sem).wait()

    iota = lax.iota(jnp.int32, 16)

    def compute(c, buf):
        scores_v[pl.ds(0, 16)] = buf[0, pl.ds(0, 16)]
        return

        def sstep(s, accs):
            r = rel_v[pl.ds(s * 16, 16)]
            return tuple(
                accs[e] + buf[e, pl.ds(s * 16, 16)] * r
                * buf[_C + e, pl.ds(s * 16, 16)]
                for e in range(_C)
            )
        accs = lax.fori_loop(
            0, _NSLICE, sstep,
            tuple(jnp.zeros((16,), jnp.float32) for _ in range(_C)))
        tot = jnp.zeros((16,), jnp.float32)
        for e in range(_C):
            tot = jnp.where(iota == e, jnp.sum(accs[e]), tot)
        scores_v[pl.ds(c * _C, _C)] = tot

    for j in range(_NBUF):
        start(j, bufs.at[j], sems.at[j])

    def step(k, carry):
        for j in range(_NBUF):
            c = _NBUF * k + j
            drain(bufs.at[j], sems.at[j])
            compute(c, bufs.at[j])

            @pl.when(c + _NBUF < _NCHUNK)
            def _():
                start(c + _NBUF, bufs.at[j], sems.at[j])
        return carry

    lax.fori_loop(0, _NCHUNK // _NBUF, step, 0)

    pltpu.sync_copy(scores_v, out_hbm.at[pl.ds(wid * _EPW, _EPW)])


@jax.jit
def kernel(x, edge_index, rel):
    ei = edge_index.astype(jnp.int32).reshape(2, _NW, _NCHUNK, _C)
    idx = jnp.concatenate([ei[0], ei[1]], axis=-1)
    mesh = plsc.VectorSubcoreMesh(
        core_axis_name="c", subcore_axis_name="s",
        num_cores=_NC, num_subcores=_NS)
    x = x.reshape(100000, 512)
    idx = idx * 2
    f = pl.kernel(
        _body,
        out_type=jax.ShapeDtypeStruct((_N_EDGES,), jnp.float32),
        mesh=mesh,
        compiler_params=pltpu.CompilerParams(needs_layout_passes=False),
        scratch_types=[
            pltpu.VMEM((_NCHUNK, 2 * _C), jnp.int32),
            pltpu.VMEM((_D,), jnp.float32),
            pltpu.VMEM((_NBUF, 2 * _C, 512), jnp.float32),
            pltpu.VMEM((_EPW,), jnp.float32),
            pltpu.SemaphoreType.DMA((_NBUF,)),
        ],
    )
    return f(x, idx, rel)


# speedup vs baseline: 1.7559x; 1.7559x over previous
import functools

import jax
import jax.numpy as jnp
from jax import lax
from jax.experimental import pallas as pl
from jax.experimental.pallas import tpu as pltpu
from jax.experimental.pallas import tpu_sc as plsc

_N_EDGES = 65536
_D = 1024
_NC = 2
_NS = 16
_NW = _NC * _NS
_EPW = _N_EDGES // _NW
_C = 16
_NCHUNK = _EPW // _C
_NBUF = 3
_NSLICE = _D // 16


def _body(x_hbm, idx_hbm, rel_hbm, out_hbm,
          idx_v, rel_v, bufs, scores_v, sems):
    wid = lax.axis_index("s") * _NC + lax.axis_index("c")

    pltpu.sync_copy(idx_hbm.at[wid], idx_v)
    pltpu.sync_copy(rel_hbm, rel_v)

    def start(c, buf, sem):
        pltpu.make_async_copy(x_hbm.at[idx_v.at[c]], buf, sem).start()

    def drain(buf, sem):
        pltpu.make_async_copy(x_hbm.at[idx_v.at[0]], buf, sem).wait()

    iota = lax.iota(jnp.int32, 16)

    def compute(c, buf):
        def sstep(s, accs):
            r = rel_v[pl.ds(s * 16, 16)]
            return tuple(
                accs[e] + buf[e, pl.ds(s * 16, 16)] * r
                * buf[_C + e, pl.ds(s * 16, 16)]
                for e in range(_C)
            )
        accs = lax.fori_loop(
            0, _NSLICE, sstep,
            tuple(jnp.zeros((16,), jnp.float32) for _ in range(_C)))
        tot = jnp.zeros((16,), jnp.float32)
        for e in range(_C):
            tot = jnp.where(iota == e, jnp.sum(accs[e]), tot)
        scores_v[pl.ds(c * _C, _C)] = tot

    for j in range(_NBUF):
        start(j, bufs.at[j], sems.at[j])

    _NFULL = _NCHUNK // _NBUF

    def step(k, carry):
        for j in range(_NBUF):
            c = _NBUF * k + j
            drain(bufs.at[j], sems.at[j])
            compute(c, bufs.at[j])

            @pl.when(c + _NBUF < _NCHUNK)
            def _():
                start(c + _NBUF, bufs.at[j], sems.at[j])
        return carry

    lax.fori_loop(0, _NFULL, step, 0)

    for j in range(_NCHUNK - _NFULL * _NBUF):
        c = _NFULL * _NBUF + j
        drain(bufs.at[j], sems.at[j])
        compute(c, bufs.at[j])

    pltpu.sync_copy(scores_v, out_hbm.at[pl.ds(wid * _EPW, _EPW)])


@jax.jit
def kernel(x, edge_index, rel):
    ei = edge_index.astype(jnp.int32).reshape(2, _NW, _NCHUNK, _C)
    idx = jnp.concatenate([ei[0], ei[1]], axis=-1)
    mesh = plsc.VectorSubcoreMesh(
        core_axis_name="c", subcore_axis_name="s",
        num_cores=_NC, num_subcores=_NS)
    f = pl.kernel(
        _body,
        out_type=jax.ShapeDtypeStruct((_N_EDGES,), jnp.float32),
        mesh=mesh,
        compiler_params=pltpu.CompilerParams(needs_layout_passes=False),
        scratch_types=[
            pltpu.VMEM((_NCHUNK, 2 * _C), jnp.int32),
            pltpu.VMEM((_D,), jnp.float32),
            pltpu.VMEM((_NBUF, 2 * _C, _D), jnp.float32),
            pltpu.VMEM((_EPW,), jnp.float32),
            pltpu.SemaphoreType.DMA((_NBUF,)),
        ],
    )
    return f(x, idx, rel)
